# baseline (speedup 1.0000x reference)
import jax
import jax.numpy as jnp
from jax.experimental import pallas as pl
from jax.experimental.pallas import tpu as pltpu
from jax.experimental.pallas import tpu_sc as plsc

_SPREAD = 8192


def _sc_gather(table, ids, n, embed):
    b, s = ids.shape
    rows_per_block = 4
    mesh = plsc.VectorSubcoreMesh(core_axis_name="c", subcore_axis_name="s")

    @pl.kernel(
        out_type=jax.ShapeDtypeStruct((n, embed), jnp.float32),
        mesh=mesh,
        compiler_params=pltpu.CompilerParams(use_tc_tiling_on_sc=False),
    )
    def gather_kernel(table_hbm, ids_hbm, out_hbm):
        def body(i_vmem, o_vmem):
            for r in range(rows_per_block):
                pltpu.sync_copy(
                    table_hbm.at[i_vmem.at[r]],
                    o_vmem.at[pl.ds(r * s, s)],
                )

        pltpu.emit_pipeline(
            body,
            grid=(b // rows_per_block,),
            in_specs=[pl.BlockSpec((rows_per_block, s), lambda i: (i, 0))],
            out_specs=[pl.BlockSpec((rows_per_block * s, embed), lambda i: (i, 0))],
            core_axis_name=("c", "s"),
            dimension_semantics=(pltpu.PARALLEL,),
        )(ids_hbm, out_hbm)

    return gather_kernel(table, ids)


def _tc_body(g_hbm, p_ref, ga_ref, be_ref, o_ref, g_vmem, sem):
    rows, lanes = o_ref.shape
    i = pl.program_id(0)
    cp = pltpu.make_async_copy(g_hbm.at[pl.ds(i * rows, rows), :], g_vmem, sem)
    cp.start()
    cp.wait()
    prow = p_ref.shape[0]
    embed = lanes // 2
    x = g_vmem[...] + jnp.tile(p_ref[...], (rows // prow, 1))
    li = jax.lax.broadcasted_iota(jnp.int32, (lanes, lanes), 0) // embed
    lj = jax.lax.broadcasted_iota(jnp.int32, (lanes, lanes), 1) // embed
    bd = jnp.where(li == lj, 1.0 / embed, 0.0).astype(jnp.float32)
    mean = jax.lax.dot(x, bd, preferred_element_type=jnp.float32)
    meansq = jax.lax.dot(x * x, bd, preferred_element_type=jnp.float32)
    var = meansq - mean * mean
    o_ref[...] = (x - mean) * jax.lax.rsqrt(var + 1e-5) * ga_ref[...] + be_ref[...]


def kernel(input_ids, mask, token_table, pos_table, mask_embedding, gamma, beta):
    b, s = input_ids.shape
    vocab, embed = token_table.shape
    n = b * s

    table_ext = jnp.concatenate(
        [token_table, jnp.broadcast_to(mask_embedding.reshape(1, embed), (_SPREAD, embed))],
        axis=0,
    )
    lin = jax.lax.broadcasted_iota(jnp.int32, (b, s), 0) * s + jax.lax.broadcasted_iota(
        jnp.int32, (b, s), 1
    )
    ids = jnp.where(mask != 0, vocab + (lin & (_SPREAD - 1)), input_ids.astype(jnp.int32))

    gathered = _sc_gather(table_ext, ids, n, embed)
    gp = gathered.reshape(n // 2, 2 * embed)

    pos2 = pos_table[:s].reshape(s // 2, 2 * embed)
    ga2 = jnp.tile(gamma, 2).reshape(1, 2 * embed)
    be2 = jnp.tile(beta, 2).reshape(1, 2 * embed)

    bb = 64
    rows = bb * s // 2
    out = pl.pallas_call(
        _tc_body,
        grid=(b // bb,),
        compiler_params=pltpu.CompilerParams(dimension_semantics=("parallel",)),
        in_specs=[
            pl.BlockSpec(memory_space=pl.MemorySpace.ANY),
            pl.BlockSpec((s // 2, 2 * embed), lambda i: (0, 0)),
            pl.BlockSpec((1, 2 * embed), lambda i: (0, 0)),
            pl.BlockSpec((1, 2 * embed), lambda i: (0, 0)),
        ],
        out_specs=pl.BlockSpec((rows, 2 * embed), lambda i: (i, 0)),
        out_shape=jax.ShapeDtypeStruct((n // 2, 2 * embed), jnp.float32),
        scratch_shapes=[
            pltpu.VMEM((rows, 2 * embed), jnp.float32),
            pltpu.SemaphoreType.DMA,
        ],
    )(gp, pos2, ga2, be2)
    return out.reshape(b, s, embed)


# speedup vs baseline: 1.1013x; 1.1013x over previous
import jax
import jax.numpy as jnp
from jax.experimental import pallas as pl
from jax.experimental.pallas import tpu as pltpu
from jax.experimental.pallas import tpu_sc as plsc

_SPREAD = 8192


def _sc_gather(table, ids, n, embed):
    b, s = ids.shape
    rows_per_block = 4
    mesh = plsc.VectorSubcoreMesh(core_axis_name="c", subcore_axis_name="s")

    @pl.kernel(
        out_type=jax.ShapeDtypeStruct((n, embed), jnp.float32),
        mesh=mesh,
        compiler_params=pltpu.CompilerParams(use_tc_tiling_on_sc=False),
    )
    def gather_kernel(table_hbm, ids_hbm, out_hbm):
        def body(i_vmem, o_vmem):
            for r in range(rows_per_block):
                pltpu.sync_copy(
                    table_hbm.at[i_vmem.at[r]],
                    o_vmem.at[pl.ds(r * s, s)],
                )

        pltpu.emit_pipeline(
            body,
            grid=(b // rows_per_block,),
            in_specs=[pl.BlockSpec((rows_per_block, s), lambda i: (i, 0))],
            out_specs=[pl.BlockSpec((rows_per_block * s, embed), lambda i: (i, 0))],
            core_axis_name=("c", "s"),
            dimension_semantics=(pltpu.PARALLEL,),
        )(ids_hbm, out_hbm)

    return gather_kernel(table, ids)


def _tc_body(g_ref, p_ref, ga_ref, be_ref, o_ref):
    rows, lanes = g_ref.shape
    prow = p_ref.shape[0]
    embed = lanes // 2
    x = g_ref[...] + jnp.tile(p_ref[...], (rows // prow, 1))
    li = jax.lax.broadcasted_iota(jnp.int32, (lanes, lanes), 0) // embed
    lj = jax.lax.broadcasted_iota(jnp.int32, (lanes, lanes), 1) // embed
    bd = jnp.where(li == lj, 1.0 / embed, 0.0).astype(jnp.float32)
    mean = jax.lax.dot(x, bd, preferred_element_type=jnp.float32)
    meansq = jax.lax.dot(x * x, bd, preferred_element_type=jnp.float32)
    var = meansq - mean * mean
    o_ref[...] = (x - mean) * jax.lax.rsqrt(var + 1e-5) * ga_ref[...] + be_ref[...]


def kernel(input_ids, mask, token_table, pos_table, mask_embedding, gamma, beta):
    b, s = input_ids.shape
    vocab, embed = token_table.shape
    n = b * s

    table_ext = jnp.concatenate(
        [token_table, jnp.broadcast_to(mask_embedding.reshape(1, embed), (_SPREAD, embed))],
        axis=0,
    )
    lin = jax.lax.broadcasted_iota(jnp.int32, (b, s), 0) * s + jax.lax.broadcasted_iota(
        jnp.int32, (b, s), 1
    )
    ids = jnp.where(mask != 0, vocab + (lin & (_SPREAD - 1)), input_ids.astype(jnp.int32))

    gathered = _sc_gather(table_ext, ids, n, embed)
    gp = gathered.reshape(n // 2, 2 * embed)

    pos2 = pos_table[:s].reshape(s // 2, 2 * embed)
    ga2 = jnp.tile(gamma, 2).reshape(1, 2 * embed)
    be2 = jnp.tile(beta, 2).reshape(1, 2 * embed)

    bb = 64
    rows = bb * s // 2
    out = pl.pallas_call(
        _tc_body,
        grid=(b // bb,),
        compiler_params=pltpu.CompilerParams(dimension_semantics=("parallel",)),
        in_specs=[
            pl.BlockSpec((rows, 2 * embed), lambda i: (i, 0)),
            pl.BlockSpec((s // 2, 2 * embed), lambda i: (0, 0)),
            pl.BlockSpec((1, 2 * embed), lambda i: (0, 0)),
            pl.BlockSpec((1, 2 * embed), lambda i: (0, 0)),
        ],
        out_specs=pl.BlockSpec((rows, 2 * embed), lambda i: (i, 0)),
        out_shape=jax.ShapeDtypeStruct((n // 2, 2 * embed), jnp.float32),
    )(gp, pos2, ga2, be2)
    return out.reshape(b, s, embed)


# speedup vs baseline: 1.1154x; 1.0128x over previous
import jax
import jax.numpy as jnp
from jax.experimental import pallas as pl
from jax.experimental.pallas import tpu as pltpu
from jax.experimental.pallas import tpu_sc as plsc

_SPREAD = 8192


def _sc_gather(table, ids, n, embed):
    b, s = ids.shape
    rows_per_block = 4
    mesh = plsc.VectorSubcoreMesh(core_axis_name="c", subcore_axis_name="s")

    @pl.kernel(
        out_type=jax.ShapeDtypeStruct((n, embed), jnp.float32),
        mesh=mesh,
        compiler_params=pltpu.CompilerParams(use_tc_tiling_on_sc=False),
    )
    def gather_kernel(table_hbm, ids_hbm, out_hbm):
        def body(i_vmem, o_vmem):
            for r in range(rows_per_block):
                pltpu.sync_copy(
                    table_hbm.at[i_vmem.at[r]],
                    o_vmem.at[pl.ds(r * s, s)],
                )

        pltpu.emit_pipeline(
            body,
            grid=(b // rows_per_block,),
            in_specs=[pl.BlockSpec((rows_per_block, s), lambda i: (i, 0))],
            out_specs=[pl.BlockSpec((rows_per_block * s, embed), lambda i: (i, 0))],
            core_axis_name=("c", "s"),
            dimension_semantics=(pltpu.PARALLEL,),
        )(ids_hbm, out_hbm)

    return gather_kernel(table, ids)


def _tc_body(g_ref, p_ref, ga_ref, be_ref, o_ref):
    rows, lanes = g_ref.shape
    prow = p_ref.shape[0]
    embed = lanes // 2
    x = g_ref[...] + jnp.tile(p_ref[...], (rows // prow, 1))
    li = jax.lax.broadcasted_iota(jnp.int32, (lanes, lanes), 0) // embed
    lj = jax.lax.broadcasted_iota(jnp.int32, (lanes, lanes), 1) // embed
    bd = jnp.where(li == lj, 1.0 / embed, 0.0).astype(jnp.float32)
    mean = jax.lax.dot(x, bd, preferred_element_type=jnp.float32)
    meansq = jax.lax.dot(x * x, bd, preferred_element_type=jnp.float32)
    var = meansq - mean * mean
    o_ref[...] = (x - mean) * jax.lax.rsqrt(var + 1e-5) * ga_ref[...] + be_ref[...]


def kernel(input_ids, mask, token_table, pos_table, mask_embedding, gamma, beta):
    b, s = input_ids.shape
    vocab, embed = token_table.shape
    n = b * s

    table_ext = jnp.concatenate(
        [token_table, jnp.broadcast_to(mask_embedding.reshape(1, embed), (_SPREAD, embed))],
        axis=0,
    )
    lin = jax.lax.broadcasted_iota(jnp.int32, (b, s), 0) * s + jax.lax.broadcasted_iota(
        jnp.int32, (b, s), 1
    )
    ids = jnp.where(mask != 0, vocab + (lin & (_SPREAD - 1)), input_ids.astype(jnp.int32))

    gathered = _sc_gather(table_ext, ids, n, embed)
    gp = gathered.reshape(n // 2, 2 * embed)

    pos2 = pos_table[:s].reshape(s // 2, 2 * embed)
    ga2 = jnp.tile(gamma, 2).reshape(1, 2 * embed)
    be2 = jnp.tile(beta, 2).reshape(1, 2 * embed)

    bb = 128
    rows = bb * s // 2
    out = pl.pallas_call(
        _tc_body,
        grid=(b // bb,),
        compiler_params=pltpu.CompilerParams(dimension_semantics=("parallel",)),
        in_specs=[
            pl.BlockSpec((rows, 2 * embed), lambda i: (i, 0)),
            pl.BlockSpec((s // 2, 2 * embed), lambda i: (0, 0)),
            pl.BlockSpec((1, 2 * embed), lambda i: (0, 0)),
            pl.BlockSpec((1, 2 * embed), lambda i: (0, 0)),
        ],
        out_specs=pl.BlockSpec((rows, 2 * embed), lambda i: (i, 0)),
        out_shape=jax.ShapeDtypeStruct((n // 2, 2 * embed), jnp.float32),
    )(gp, pos2, ga2, be2)
    return out.reshape(b, s, embed)
